# Initial kernel scaffold; baseline (speedup 1.0000x reference)
import jax
import jax.numpy as jnp
from jax.experimental import pallas as pl


def kernel(points, elements, W1, b1, W2, b2, W3, b3, W4, b4):
    raise NotImplementedError("write your pallas kernel here")


# speedup vs baseline: 6.0902x; 6.0902x over previous
import jax
import jax.numpy as jnp
from jax import lax
from jax.experimental import pallas as pl
from jax.experimental.pallas import tpu as pltpu
from jax.experimental.pallas import tpu_sc as plsc

N_POINTS = 100000
N_ELEMENTS = 200000
ELEMENT_SIZE = 3
DIM = 2

NC, NS, LANES = 2, 16, 16
NW = NC * NS
CHUNK_E = 128
CHUNK_R = CHUNK_E * ELEMENT_SIZE
N_CHUNKS = 50
E_PAD = NW * N_CHUNKS * CHUNK_E
R_PER_TILE = N_CHUNKS * CHUNK_R
IDX_ROWS = R_PER_TILE // 128
ACC_N = 100352
DUMMY = N_POINTS
OUT_SLICE = ACC_N // NS

W1_OFF = 0
B1_OFF = 48
W2_OFF = 56
B2_OFF = 120
W3_OFF = 128
B3_OFF = 192
W4_OFF = 200
B4_OFF = 224
W_LEN = 240


def _sig(x):
    return 1.0 / (1.0 + jnp.exp(-x))


def _sc_body(pts_hbm, sidx_hbm, wbuf_hbm, zeros_hbm, out_hbm,
             sidx_v, gat_v, vals_v, wv, acc_sh, sem):
    c = lax.axis_index("c")
    s = lax.axis_index("s")
    wid = c * NS + s

    @pl.when(s == 0)
    def _():
        pltpu.sync_copy(zeros_hbm, acc_sh)

    pltpu.sync_copy(sidx_hbm.at[wid], sidx_v)
    pltpu.sync_copy(wbuf_hbm, wv)
    plsc.subcore_barrier()

    w = []
    for b in range(0, W_LEN, LANES):
        vec = wv[pl.ds(b, LANES)]
        for j in range(LANES):
            if b + j < B4_OFF + ELEMENT_SIZE:
                w.append(vec[j])
    iota = lax.iota(jnp.int32, LANES)
    i3 = iota * 3
    col0 = jnp.zeros((LANES,), jnp.int32)
    col1 = col0 + 1

    def chunk(ci, carry):
        base = ci * (CHUNK_R // 128)
        cps = []
        for k in range(CHUNK_R // 128):
            cp = pltpu.async_copy(
                pts_hbm.at[sidx_v.at[base + k]],
                gat_v.at[pl.ds(k * 128, 128)], sem)
            cps.append(cp)
        for cp in cps:
            cp.wait()

        for g in range(CHUNK_E // LANES):
            ins = []
            for v in range(ELEMENT_SIZE):
                rows = i3 + (g * 3 * LANES + v)
                ins.append(plsc.load_gather(gat_v, [rows, col0]))
                ins.append(plsc.load_gather(gat_v, [rows, col1]))
            h1 = [_sig(sum(ins[i] * w[W1_OFF + i * 8 + j] for i in range(6))
                       + w[B1_OFF + j]) for j in range(8)]
            h2 = [_sig(sum(h1[i] * w[W2_OFF + i * 8 + j] for i in range(8))
                       + w[B2_OFF + j]) for j in range(8)]
            h3 = [_sig(sum(h2[i] * w[W3_OFF + i * 8 + j] for i in range(8))
                       + w[B3_OFF + j]) for j in range(8)]
            for v in range(ELEMENT_SIZE):
                o = _sig(sum(h3[i] * w[W4_OFF + i * 3 + v] for i in range(8))
                         + w[B4_OFF + v])
                rows = i3 + (g * 3 * LANES + v)
                plsc.store_scatter(vals_v, [rows], o)

        for k in range(CHUNK_R // 128):
            pltpu.sync_copy(vals_v.at[pl.ds(k * 128, 128)],
                            acc_sh.at[sidx_v.at[base + k]], add=True)
        return carry

    lax.fori_loop(0, N_CHUNKS, chunk, 0)
    plsc.subcore_barrier()

    pltpu.sync_copy(acc_sh.at[pl.ds(s * OUT_SLICE, OUT_SLICE)],
                    out_hbm.at[c, pl.ds(s * OUT_SLICE, OUT_SLICE)])


@jax.jit
def _sc_call(pts, sidx, wbuf, zeros):
    mesh = plsc.VectorSubcoreMesh(core_axis_name="c", subcore_axis_name="s")
    return pl.kernel(
        _sc_body,
        out_type=jax.ShapeDtypeStruct((NC, ACC_N), jnp.float32),
        mesh=mesh,
        scratch_types=[
            pltpu.VMEM((IDX_ROWS, 128), jnp.int32),
            pltpu.VMEM((CHUNK_R, DIM), jnp.float32),
            pltpu.VMEM((CHUNK_R,), jnp.float32),
            pltpu.VMEM((W_LEN,), jnp.float32),
            pltpu.VMEM_SHARED((ACC_N,), jnp.float32),
            pltpu.SemaphoreType.DMA,
        ],
        compiler_params=pltpu.CompilerParams(
            needs_layout_passes=False, use_tc_tiling_on_sc=False),
    )(pts, sidx, wbuf, zeros)


def _combine_body(p_ref, o_ref):
    o_ref[...] = p_ref[0] + p_ref[1]


@jax.jit
def _combine(partials):
    p = partials.reshape(NC, ACC_N // 128, 128)
    out = pl.pallas_call(
        _combine_body,
        out_shape=jax.ShapeDtypeStruct((ACC_N // 128, 128), jnp.float32),
    )(p)
    return out.reshape(-1)[:N_POINTS]


def kernel(points, elements, W1, b1, W2, b2, W3, b3, W4, b4):
    pts = jnp.concatenate(
        [points, jnp.zeros((1, DIM), jnp.float32)], axis=0)
    flat = elements.reshape(-1).astype(jnp.int32)
    pad = jnp.full((E_PAD * ELEMENT_SIZE - flat.shape[0],), DUMMY, jnp.int32)
    sidx = jnp.concatenate([flat, pad]).reshape(NW, IDX_ROWS, 128)
    wbuf = jnp.zeros((W_LEN,), jnp.float32)
    wbuf = wbuf.at[W1_OFF:W1_OFF + 48].set(W1.reshape(-1))
    wbuf = wbuf.at[B1_OFF:B1_OFF + 8].set(b1)
    wbuf = wbuf.at[W2_OFF:W2_OFF + 64].set(W2.reshape(-1))
    wbuf = wbuf.at[B2_OFF:B2_OFF + 8].set(b2)
    wbuf = wbuf.at[W3_OFF:W3_OFF + 64].set(W3.reshape(-1))
    wbuf = wbuf.at[B3_OFF:B3_OFF + 8].set(b3)
    wbuf = wbuf.at[W4_OFF:W4_OFF + 24].set(W4.reshape(-1))
    wbuf = wbuf.at[B4_OFF:B4_OFF + 3].set(b4)
    zeros = jnp.zeros((ACC_N,), jnp.float32)
    partials = _sc_call(pts, sidx, wbuf, zeros)
    return _combine(partials)


# speedup vs baseline: 7.0676x; 1.1605x over previous
import jax
import jax.numpy as jnp
from jax import lax
from jax.experimental import pallas as pl
from jax.experimental.pallas import tpu as pltpu
from jax.experimental.pallas import tpu_sc as plsc

N_POINTS = 100000
N_ELEMENTS = 200000
ELEMENT_SIZE = 3
DIM = 2

NC, NS, LANES = 2, 16, 16
NW = NC * NS
CHUNK_E = 128
CHUNK_R = CHUNK_E * ELEMENT_SIZE
N_CHUNKS = 50
E_PAD = NW * N_CHUNKS * CHUNK_E
R_PER_TILE = N_CHUNKS * CHUNK_R
IDX_ROWS = R_PER_TILE // 128
ACC_N = 100352
DUMMY = N_POINTS
OUT_SLICE = ACC_N // NS

W1_OFF = 0
B1_OFF = 48
W2_OFF = 56
B2_OFF = 120
W3_OFF = 128
B3_OFF = 192
W4_OFF = 200
B4_OFF = 224
W_LEN = 240


def _sig(x):
    return 1.0 / (1.0 + jnp.exp(-x))


def _sc_body(pts_hbm, sidx_hbm, wbuf_hbm, zeros_hbm, out_hbm,
             sidx_v, gat_v, vals_v, wv, acc_sh, sem_g, sem_s):
    c = lax.axis_index("c")
    s = lax.axis_index("s")
    wid = c * NS + s

    @pl.when(s == 0)
    def _():
        pltpu.sync_copy(zeros_hbm, acc_sh)

    pltpu.sync_copy(sidx_hbm.at[wid], sidx_v)
    pltpu.sync_copy(wbuf_hbm, wv)
    plsc.subcore_barrier()

    w = []
    for b in range(0, W_LEN, LANES):
        vec = wv[pl.ds(b, LANES)]
        for j in range(LANES):
            if b + j < B4_OFF + ELEMENT_SIZE:
                w.append(vec[j])
    iota = lax.iota(jnp.int32, LANES)
    i3 = iota * 3
    col0 = jnp.zeros((LANES,), jnp.int32)
    col1 = col0 + 1

    NK = CHUNK_R // 128

    def fire_gathers(ci, par):
        base = ci * NK
        for k in range(NK):
            pltpu.async_copy(
                pts_hbm.at[sidx_v.at[base + k]],
                gat_v.at[par].at[pl.ds(k * 128, 128)], sem_g)

    def drain_gathers(par):
        for k in range(NK):
            pltpu.make_async_copy(
                pts_hbm.at[sidx_v.at[k]],
                gat_v.at[par].at[pl.ds(k * 128, 128)], sem_g).wait()

    def fire_scatters(ci, par):
        base = ci * NK
        for k in range(NK):
            pltpu.async_copy(
                vals_v.at[par].at[pl.ds(k * 128, 128)],
                acc_sh.at[sidx_v.at[base + k]], sem_s, add=True)

    def drain_scatters(par):
        for k in range(NK):
            pltpu.make_async_copy(
                vals_v.at[par].at[pl.ds(k * 128, 128)],
                acc_sh.at[sidx_v.at[k]], sem_s).wait()

    fire_gathers(0, 0)

    def chunk(ci, carry):
        par = lax.rem(ci, 2)
        drain_gathers(par)

        @pl.when(ci < N_CHUNKS - 1)
        def _():
            fire_gathers(ci + 1, 1 - par)

        @pl.when(ci >= 2)
        def _():
            drain_scatters(par)

        gref = gat_v.at[par]
        vref = vals_v.at[par]
        for g in range(CHUNK_E // LANES):
            ins = []
            for v in range(ELEMENT_SIZE):
                rows = i3 + (g * 3 * LANES + v)
                ins.append(plsc.load_gather(gref, [rows, col0]))
                ins.append(plsc.load_gather(gref, [rows, col1]))
            h1 = [_sig(sum(ins[i] * w[W1_OFF + i * 8 + j] for i in range(6))
                       + w[B1_OFF + j]) for j in range(8)]
            h2 = [_sig(sum(h1[i] * w[W2_OFF + i * 8 + j] for i in range(8))
                       + w[B2_OFF + j]) for j in range(8)]
            h3 = [_sig(sum(h2[i] * w[W3_OFF + i * 8 + j] for i in range(8))
                       + w[B3_OFF + j]) for j in range(8)]
            for v in range(ELEMENT_SIZE):
                o = _sig(sum(h3[i] * w[W4_OFF + i * 3 + v] for i in range(8))
                         + w[B4_OFF + v])
                rows = i3 + (g * 3 * LANES + v)
                plsc.store_scatter(vref, [rows], o)

        fire_scatters(ci, par)
        return carry

    lax.fori_loop(0, N_CHUNKS, chunk, 0)
    drain_scatters(0)
    drain_scatters(1)
    plsc.subcore_barrier()

    pltpu.sync_copy(acc_sh.at[pl.ds(s * OUT_SLICE, OUT_SLICE)],
                    out_hbm.at[c, pl.ds(s * OUT_SLICE, OUT_SLICE)])


@jax.jit
def _sc_call(pts, sidx, wbuf, zeros):
    mesh = plsc.VectorSubcoreMesh(core_axis_name="c", subcore_axis_name="s")
    return pl.kernel(
        _sc_body,
        out_type=jax.ShapeDtypeStruct((NC, ACC_N), jnp.float32),
        mesh=mesh,
        scratch_types=[
            pltpu.VMEM((IDX_ROWS, 128), jnp.int32),
            pltpu.VMEM((2, CHUNK_R, DIM), jnp.float32),
            pltpu.VMEM((2, CHUNK_R), jnp.float32),
            pltpu.VMEM((W_LEN,), jnp.float32),
            pltpu.VMEM_SHARED((ACC_N,), jnp.float32),
            pltpu.SemaphoreType.DMA,
            pltpu.SemaphoreType.DMA,
        ],
        compiler_params=pltpu.CompilerParams(
            needs_layout_passes=False, use_tc_tiling_on_sc=False),
    )(pts, sidx, wbuf, zeros)


def _combine_body(p_ref, o_ref):
    o_ref[...] = p_ref[0] + p_ref[1]


@jax.jit
def _combine(partials):
    p = partials.reshape(NC, ACC_N // 128, 128)
    out = pl.pallas_call(
        _combine_body,
        out_shape=jax.ShapeDtypeStruct((ACC_N // 128, 128), jnp.float32),
    )(p)
    return out.reshape(-1)[:N_POINTS]


def kernel(points, elements, W1, b1, W2, b2, W3, b3, W4, b4):
    pts = jnp.concatenate(
        [points, jnp.zeros((1, DIM), jnp.float32)], axis=0)
    flat = elements.reshape(-1).astype(jnp.int32)
    pad = jnp.full((E_PAD * ELEMENT_SIZE - flat.shape[0],), DUMMY, jnp.int32)
    sidx = jnp.concatenate([flat, pad]).reshape(NW, IDX_ROWS, 128)
    wbuf = jnp.zeros((W_LEN,), jnp.float32)
    wbuf = wbuf.at[W1_OFF:W1_OFF + 48].set(W1.reshape(-1))
    wbuf = wbuf.at[B1_OFF:B1_OFF + 8].set(b1)
    wbuf = wbuf.at[W2_OFF:W2_OFF + 64].set(W2.reshape(-1))
    wbuf = wbuf.at[B2_OFF:B2_OFF + 8].set(b2)
    wbuf = wbuf.at[W3_OFF:W3_OFF + 64].set(W3.reshape(-1))
    wbuf = wbuf.at[B3_OFF:B3_OFF + 8].set(b3)
    wbuf = wbuf.at[W4_OFF:W4_OFF + 24].set(W4.reshape(-1))
    wbuf = wbuf.at[B4_OFF:B4_OFF + 3].set(b4)
    zeros = jnp.zeros((ACC_N,), jnp.float32)
    partials = _sc_call(pts, sidx, wbuf, zeros)
    return _combine(partials)
